# Initial kernel scaffold; baseline (speedup 1.0000x reference)
import functools

import jax
import jax.numpy as jnp
from jax import lax
from jax.experimental import pallas as pl
from jax.experimental.pallas import tpu as pltpu

INPUT_DIMS = 1024
INDEX_DIMS = 128
SZ0 = 4096
BRANCH = 16
K = 16
ROWS = 2048
ROW_BLK = 256


def _tc_body(x_ref, w_ref, b_ref, k0_ref, logits_ref, q1_ref):
    q = jnp.dot(x_ref[...], w_ref[...], preferred_element_type=jnp.float32)
    q = q + b_ref[...]
    q0 = q[:, :INDEX_DIMS]
    q1_ref[...] = q[:, INDEX_DIMS:]
    lg = lax.dot_general(q0, k0_ref[...], (((1,), (1,)), ((), ())),
                         preferred_element_type=jnp.float32)
    logits_ref[...] = lg / jnp.sqrt(jnp.float32(INDEX_DIMS))


def _tc_stage(x, W, b, key0):
    grid = (ROWS // ROW_BLK,)
    return pl.pallas_call(
        _tc_body,
        grid=grid,
        in_specs=[
            pl.BlockSpec((ROW_BLK, INPUT_DIMS), lambda i: (i, 0)),
            pl.BlockSpec((INPUT_DIMS, 2 * INDEX_DIMS), lambda i: (0, 0)),
            pl.BlockSpec((1, 2 * INDEX_DIMS), lambda i: (0, 0)),
            pl.BlockSpec((SZ0, INDEX_DIMS), lambda i: (0, 0)),
        ],
        out_specs=[
            pl.BlockSpec((ROW_BLK, SZ0), lambda i: (i, 0)),
            pl.BlockSpec((ROW_BLK, INDEX_DIMS), lambda i: (i, 0)),
        ],
        out_shape=[
            jax.ShapeDtypeStruct((ROWS, SZ0), jnp.float32),
            jax.ShapeDtypeStruct((ROWS, INDEX_DIMS), jnp.float32),
        ],
    )(x, W, b.reshape(1, -1), key0)


def kernel(input_hiddens, W, b, key0, key1):
    logits0, q1 = _tc_stage(input_hiddens, W, b, key0)
    attn = jax.nn.softmax(logits0, axis=-1)
    attn_topk_val, attn_topk_ind = jax.lax.top_k(attn, K)
    attn_topk_val = attn_topk_val / jnp.sum(attn_topk_val, axis=-1, keepdims=True)
    kk = jnp.take(key1, attn_topk_ind, axis=0)
    logits = jnp.matmul(q1[:, None, None, :], jnp.swapaxes(kk, -1, -2))
    logits = jnp.squeeze(logits, -2) / jnp.sqrt(jnp.float32(INDEX_DIMS))
    attn = attn_topk_val[..., None] * jax.nn.softmax(logits, axis=-1)
    attn = attn.reshape(attn.shape[:-2] + (-1,))
    attn_topk_val, attn_topk_ind = jax.lax.top_k(attn, K)
    attn_topk_val = attn_topk_val / jnp.sum(attn_topk_val, axis=-1, keepdims=True)
    return (attn_topk_val, attn_topk_ind)


# speedup vs baseline: 3.7701x; 3.7701x over previous
import functools

import jax
import jax.numpy as jnp
from jax import lax
from jax.experimental import pallas as pl
from jax.experimental.pallas import tpu as pltpu
from jax.experimental.pallas import tpu_sc as plsc

INPUT_DIMS = 1024
D = 128
SZ0 = 4096
BRANCH = 16
K = 16
ROWS = 2048
ROW_BLK = 256
NW = 32
RPW = ROWS // NW


def _halv16(x):
    x = x[..., :8] + x[..., 8:16]
    x = x[..., :4] + x[..., 4:8]
    x = x[..., :2] + x[..., 2:4]
    return x[..., 0:1] + x[..., 1:2]


def _p1_body(x_ref, w_ref, b_ref, k0_ref, e_ref, q1_ref):
    q = jnp.dot(x_ref[...], w_ref[...], preferred_element_type=jnp.float32)
    q = q + b_ref[...]
    q0 = q[:, :D]
    q1_ref[...] = q[:, D:]
    lg = lax.dot_general(q0, k0_ref[...], (((1,), (1,)), ((), ())),
                         preferred_element_type=jnp.float32)
    lg = lg / jnp.sqrt(jnp.float32(D))
    m = jnp.max(lg, axis=-1, keepdims=True)
    e_ref[...] = jnp.exp(lg - m)


def _p1(x, W, b, key0):
    return pl.pallas_call(
        _p1_body,
        grid=(ROWS // ROW_BLK,),
        in_specs=[
            pl.BlockSpec((ROW_BLK, INPUT_DIMS), lambda i: (i, 0)),
            pl.BlockSpec((INPUT_DIMS, 2 * D), lambda i: (0, 0)),
            pl.BlockSpec((1, 2 * D), lambda i: (0, 0)),
            pl.BlockSpec((SZ0, D), lambda i: (0, 0)),
        ],
        out_specs=[
            pl.BlockSpec((ROW_BLK, SZ0), lambda i: (i, 0)),
            pl.BlockSpec((ROW_BLK, D), lambda i: (i, 0)),
        ],
        out_shape=[
            jax.ShapeDtypeStruct((ROWS, SZ0), jnp.float32),
            jax.ShapeDtypeStruct((ROWS, D), jnp.float32),
        ],
    )(x, W, b.reshape(1, -1), key0)


def _p2_body(e_ref, s_ref, i0_ref, v0_ref):
    a = e_ref[...] / s_ref[...]
    iota = lax.broadcasted_iota(jnp.int32, (ROW_BLK, SZ0), 1)
    vals = []
    idxs = []
    cur = a
    for _ in range(K):
        gm = jnp.max(cur, axis=-1, keepdims=True)
        sel = jnp.where(cur == gm, iota, SZ0)
        ix = jnp.min(sel, axis=-1, keepdims=True)
        vals.append(gm)
        idxs.append(ix)
        cur = jnp.where(iota == ix, -jnp.inf, cur)
    tv = jnp.concatenate(vals, axis=-1)
    i0_ref[...] = jnp.concatenate(idxs, axis=-1)
    v0_ref[...] = tv / _halv16(tv)


def _p2(e, S):
    return pl.pallas_call(
        _p2_body,
        grid=(ROWS // ROW_BLK,),
        in_specs=[
            pl.BlockSpec((ROW_BLK, SZ0), lambda i: (i, 0)),
            pl.BlockSpec((ROW_BLK, 1), lambda i: (i, 0)),
        ],
        out_specs=[
            pl.BlockSpec((ROW_BLK, K), lambda i: (i, 0)),
            pl.BlockSpec((ROW_BLK, K), lambda i: (i, 0)),
        ],
        out_shape=[
            jax.ShapeDtypeStruct((ROWS, K), jnp.int32),
            jax.ShapeDtypeStruct((ROWS, K), jnp.float32),
        ],
    )(e, S)


def _p3_body(i0_hbm, q1_hbm, k1t_hbm, dots_hbm, idx_v, q1_v, kk_v, out_v, dsem):
    wid = lax.axis_index("s") * 2 + lax.axis_index("c")
    base = wid * RPW

    def row_body(r, carry):
        row = base + r
        pltpu.sync_copy(i0_hbm.at[row], idx_v)
        pltpu.sync_copy(q1_hbm.at[row], q1_v)
        pltpu.async_copy(k1t_hbm.at[idx_v], kk_v, dsem).wait()

        def k_body(k, carry2):
            acc = None
            for t in range(8):
                qv = q1_v[pl.ds(t * 16, 16)]
                for half in range(2):
                    g = 2 * t + half
                    ps = [kk_v[k, pl.ds((16 * t + 8 * half + m) * 16, 16)]
                          * qv[8 * half + m] for m in range(8)]
                    grp = ((ps[0] + ps[1]) + (ps[2] + ps[3])) + (
                        (ps[4] + ps[5]) + (ps[6] + ps[7]))
                    acc = grp if g == 0 else acc + grp
            out_v[r, pl.ds(k * 16, 16)] = acc
            return carry2

        lax.fori_loop(0, K, k_body, 0)
        return carry

    lax.fori_loop(0, RPW, row_body, 0)
    pltpu.sync_copy(out_v, dots_hbm.at[pl.ds(base, RPW)])


def _p3(i0, q1, k1t):
    mesh = plsc.VectorSubcoreMesh(core_axis_name="c", subcore_axis_name="s")
    f = functools.partial(
        pl.kernel,
        out_type=jax.ShapeDtypeStruct((ROWS, K * BRANCH), jnp.float32),
        mesh=mesh,
        scratch_types=[
            pltpu.VMEM((K,), jnp.int32),
            pltpu.VMEM((D,), jnp.float32),
            pltpu.VMEM((K, D * BRANCH), jnp.float32),
            pltpu.VMEM((RPW, K * BRANCH), jnp.float32),
            pltpu.SemaphoreType.DMA,
        ],
    )(_p3_body)
    return f(i0, q1, k1t)


def _p4_body(dots_ref, v0_ref, fv_ref, fi_ref):
    lg = dots_ref[...] / jnp.sqrt(jnp.float32(D))
    lg = lg.reshape(ROW_BLK, K, BRANCH)
    mx = jnp.max(lg, axis=-1, keepdims=True)
    e1 = jnp.exp(lg - mx)
    s1 = _halv16(e1)
    sm = e1 / s1
    attn = (v0_ref[...][:, :, None] * sm).reshape(ROW_BLK, K * BRANCH)
    iota = lax.broadcasted_iota(jnp.int32, (ROW_BLK, K * BRANCH), 1)
    vals = []
    idxs = []
    cur = attn
    for _ in range(K):
        gm = jnp.max(cur, axis=-1, keepdims=True)
        sel = jnp.where(cur == gm, iota, K * BRANCH)
        ix = jnp.min(sel, axis=-1, keepdims=True)
        vals.append(gm)
        idxs.append(ix)
        cur = jnp.where(iota == ix, -jnp.inf, cur)
    fv = jnp.concatenate(vals, axis=-1)
    fi_ref[...] = jnp.concatenate(idxs, axis=-1)
    fv_ref[...] = fv / _halv16(fv)


def _p4(dots, v0):
    return pl.pallas_call(
        _p4_body,
        grid=(ROWS // ROW_BLK,),
        in_specs=[
            pl.BlockSpec((ROW_BLK, K * BRANCH), lambda i: (i, 0)),
            pl.BlockSpec((ROW_BLK, K), lambda i: (i, 0)),
        ],
        out_specs=[
            pl.BlockSpec((ROW_BLK, K), lambda i: (i, 0)),
            pl.BlockSpec((ROW_BLK, K), lambda i: (i, 0)),
        ],
        out_shape=[
            jax.ShapeDtypeStruct((ROWS, K), jnp.float32),
            jax.ShapeDtypeStruct((ROWS, K), jnp.int32),
        ],
    )(dots, v0)


def kernel(input_hiddens, W, b, key0, key1):
    e, q1 = _p1(input_hiddens, W, b, key0)
    S = jnp.sum(e, axis=-1, keepdims=True)
    i0, v0 = _p2(e, S)
    k1t = key1.transpose(0, 2, 1).reshape(SZ0, D * BRANCH)
    dots = _p3(i0, q1, k1t)
    fv, fi = _p4(dots, v0)
    return (fv, fi)


# speedup vs baseline: 4.9462x; 1.3119x over previous
import functools

import jax
import jax.numpy as jnp
from jax import lax
from jax.experimental import pallas as pl
from jax.experimental.pallas import tpu as pltpu
from jax.experimental.pallas import tpu_sc as plsc

INPUT_DIMS = 1024
D = 128
SZ0 = 4096
BRANCH = 16
K = 16
ROWS = 2048
ROW_BLK = 256
NW = 32
RPW = ROWS // NW


def _halv16(x):
    x = x[..., :8] + x[..., 8:16]
    x = x[..., :4] + x[..., 4:8]
    x = x[..., :2] + x[..., 2:4]
    return x[..., 0:1] + x[..., 1:2]


def _p1_body(x_ref, w_ref, b_ref, k0_ref, e_ref, q1_ref):
    q = jnp.dot(x_ref[...], w_ref[...], preferred_element_type=jnp.float32)
    q = q + b_ref[...]
    q0 = q[:, :D]
    q1_ref[...] = q[:, D:]
    lg = lax.dot_general(q0, k0_ref[...], (((1,), (1,)), ((), ())),
                         preferred_element_type=jnp.float32)
    lg = lg / jnp.sqrt(jnp.float32(D))
    m = jnp.max(lg, axis=-1, keepdims=True)
    e_ref[...] = jnp.exp(lg - m)


def _p1(x, W, b, key0):
    return pl.pallas_call(
        _p1_body,
        grid=(ROWS // ROW_BLK,),
        in_specs=[
            pl.BlockSpec((ROW_BLK, INPUT_DIMS), lambda i: (i, 0)),
            pl.BlockSpec((INPUT_DIMS, 2 * D), lambda i: (0, 0)),
            pl.BlockSpec((1, 2 * D), lambda i: (0, 0)),
            pl.BlockSpec((SZ0, D), lambda i: (0, 0)),
        ],
        out_specs=[
            pl.BlockSpec((ROW_BLK, SZ0), lambda i: (i, 0)),
            pl.BlockSpec((ROW_BLK, D), lambda i: (i, 0)),
        ],
        out_shape=[
            jax.ShapeDtypeStruct((ROWS, SZ0), jnp.float32),
            jax.ShapeDtypeStruct((ROWS, D), jnp.float32),
        ],
    )(x, W, b.reshape(1, -1), key0)


def _p2_body(e_ref, s_ref, i0_ref, v0_ref):
    a = e_ref[...] / s_ref[...]
    iota = lax.broadcasted_iota(jnp.int32, (ROW_BLK, SZ0), 1)
    vals = []
    idxs = []
    cur = a
    for _ in range(K):
        gm = jnp.max(cur, axis=-1, keepdims=True)
        sel = jnp.where(cur == gm, iota, SZ0)
        ix = jnp.min(sel, axis=-1, keepdims=True)
        vals.append(gm)
        idxs.append(ix)
        cur = jnp.where(iota == ix, -jnp.inf, cur)
    tv = jnp.concatenate(vals, axis=-1)
    i0_ref[...] = jnp.concatenate(idxs, axis=-1)
    v0_ref[...] = tv / _halv16(tv)


def _p2(e, S):
    return pl.pallas_call(
        _p2_body,
        grid=(ROWS // ROW_BLK,),
        in_specs=[
            pl.BlockSpec((ROW_BLK, SZ0), lambda i: (i, 0)),
            pl.BlockSpec((ROW_BLK, 1), lambda i: (i, 0)),
        ],
        out_specs=[
            pl.BlockSpec((ROW_BLK, K), lambda i: (i, 0)),
            pl.BlockSpec((ROW_BLK, K), lambda i: (i, 0)),
        ],
        out_shape=[
            jax.ShapeDtypeStruct((ROWS, K), jnp.int32),
            jax.ShapeDtypeStruct((ROWS, K), jnp.float32),
        ],
    )(e, S)


def _p3_body(i0_hbm, q1_hbm, k1t_hbm, dots_hbm, idx_v, q1_v, kk0_v, kk1_v,
             out_v, sem0, sem1):
    wid = lax.axis_index("s") * 2 + lax.axis_index("c")
    base = wid * RPW
    pltpu.sync_copy(i0_hbm.at[pl.ds(base, RPW)], idx_v)
    pltpu.sync_copy(q1_hbm.at[pl.ds(base, RPW)], q1_v)

    def start(r, kk_v, sem):
        pltpu.async_copy(k1t_hbm.at[idx_v.at[r]], kk_v, sem)

    def wait(r, kk_v, sem):
        pltpu.make_async_copy(k1t_hbm.at[idx_v.at[r]], kk_v, sem).wait()

    def compute(r, kk_v):
        def k_body(k, carry2):
            acc = None
            for t in range(8):
                qv = q1_v[r, pl.ds(t * 16, 16)]
                for half in range(2):
                    g = 2 * t + half
                    ps = [kk_v[k, pl.ds((16 * t + 8 * half + m) * 16, 16)]
                          * qv[8 * half + m] for m in range(8)]
                    grp = ((ps[0] + ps[1]) + (ps[2] + ps[3])) + (
                        (ps[4] + ps[5]) + (ps[6] + ps[7]))
                    acc = grp if g == 0 else acc + grp
            out_v[r, pl.ds(k * 16, 16)] = acc
            return carry2

        lax.fori_loop(0, K, k_body, 0)

    start(0, kk0_v, sem0)

    def pair_body(i, carry):
        r0 = 2 * i
        start(r0 + 1, kk1_v, sem1)
        wait(r0, kk0_v, sem0)
        compute(r0, kk0_v)

        @pl.when(r0 + 2 < RPW)
        def _():
            start(r0 + 2, kk0_v, sem0)

        wait(r0 + 1, kk1_v, sem1)
        compute(r0 + 1, kk1_v)
        return carry

    lax.fori_loop(0, RPW // 2, pair_body, 0)
    pltpu.sync_copy(out_v, dots_hbm.at[pl.ds(base, RPW)])


def _p3(i0, q1, k1t):
    mesh = plsc.VectorSubcoreMesh(core_axis_name="c", subcore_axis_name="s")
    f = functools.partial(
        pl.kernel,
        out_type=jax.ShapeDtypeStruct((ROWS, K * BRANCH), jnp.float32),
        mesh=mesh,
        scratch_types=[
            pltpu.VMEM((RPW, K), jnp.int32),
            pltpu.VMEM((RPW, D), jnp.float32),
            pltpu.VMEM((K, D * BRANCH), jnp.float32),
            pltpu.VMEM((K, D * BRANCH), jnp.float32),
            pltpu.VMEM((RPW, K * BRANCH), jnp.float32),
            pltpu.SemaphoreType.DMA,
            pltpu.SemaphoreType.DMA,
        ],
    )(_p3_body)
    return f(i0, q1, k1t)


def _p4_body(dots_ref, v0_ref, fv_ref, fi_ref):
    lg = dots_ref[...] / jnp.sqrt(jnp.float32(D))
    lg = lg.reshape(ROW_BLK, K, BRANCH)
    mx = jnp.max(lg, axis=-1, keepdims=True)
    e1 = jnp.exp(lg - mx)
    s1 = _halv16(e1)
    sm = e1 / s1
    attn = (v0_ref[...][:, :, None] * sm).reshape(ROW_BLK, K * BRANCH)
    iota = lax.broadcasted_iota(jnp.int32, (ROW_BLK, K * BRANCH), 1)
    vals = []
    idxs = []
    cur = attn
    for _ in range(K):
        gm = jnp.max(cur, axis=-1, keepdims=True)
        sel = jnp.where(cur == gm, iota, K * BRANCH)
        ix = jnp.min(sel, axis=-1, keepdims=True)
        vals.append(gm)
        idxs.append(ix)
        cur = jnp.where(iota == ix, -jnp.inf, cur)
    fv = jnp.concatenate(vals, axis=-1)
    fi_ref[...] = jnp.concatenate(idxs, axis=-1)
    fv_ref[...] = fv / _halv16(fv)


def _p4(dots, v0):
    return pl.pallas_call(
        _p4_body,
        grid=(ROWS // ROW_BLK,),
        in_specs=[
            pl.BlockSpec((ROW_BLK, K * BRANCH), lambda i: (i, 0)),
            pl.BlockSpec((ROW_BLK, K), lambda i: (i, 0)),
        ],
        out_specs=[
            pl.BlockSpec((ROW_BLK, K), lambda i: (i, 0)),
            pl.BlockSpec((ROW_BLK, K), lambda i: (i, 0)),
        ],
        out_shape=[
            jax.ShapeDtypeStruct((ROWS, K), jnp.float32),
            jax.ShapeDtypeStruct((ROWS, K), jnp.int32),
        ],
    )(dots, v0)


def kernel(input_hiddens, W, b, key0, key1):
    e, q1 = _p1(input_hiddens, W, b, key0)
    S = jnp.sum(e, axis=-1, keepdims=True)
    i0, v0 = _p2(e, S)
    k1t = key1.transpose(0, 2, 1).reshape(SZ0, D * BRANCH)
    dots = _p3(i0, q1, k1t)
    fv, fi = _p4(dots, v0)
    return (fv, fi)
